# Initial kernel scaffold; baseline (speedup 1.0000x reference)
import jax
import jax.numpy as jnp
from jax.experimental import pallas as pl


def kernel(h, batch, vn_h, W1, bn_gamma, bn_beta, bn_mean, bn_var, W2):
    raise NotImplementedError("write your pallas kernel here")


# speedup vs baseline: 5.3332x; 5.3332x over previous
import functools

import jax
import jax.numpy as jnp
from jax import lax
from jax.experimental import pallas as pl
from jax.experimental.pallas import tpu as pltpu

N = 100000
DIM = 128
G = 128
EPS = 1e-5
R = 2000
GRID = N // R


def _pool_mlp_body(h_ref, b_ref, vn_ref, w1_ref, g_ref, be_ref, mu_ref,
                   var_ref, w2_ref, out_ref, acc_ref):
    i = pl.program_id(0)

    @pl.when(i == 0)
    def _():
        acc_ref[...] = jnp.zeros_like(acc_ref)

    ids = b_ref[0, 0, :]
    oh = (ids[:, None] == lax.broadcasted_iota(jnp.int32, (1, G), 1)
          ).astype(jnp.float32)
    acc_ref[...] += lax.dot_general(
        oh, h_ref[...], (((0,), (0,)), ((), ())),
        preferred_element_type=jnp.float32)

    @pl.when(i == GRID - 1)
    def _():
        x = acc_ref[...] + vn_ref[...]
        y = lax.dot_general(x, w1_ref[...], (((1,), (1,)), ((), ())),
                            preferred_element_type=jnp.float32)
        y = g_ref[...] * (y - mu_ref[...]) * lax.rsqrt(var_ref[...] + EPS) \
            + be_ref[...]
        y = jnp.maximum(y, 0.0)
        out_ref[...] = lax.dot_general(
            y, w2_ref[...], (((1,), (1,)), ((), ())),
            preferred_element_type=jnp.float32)


def _broadcast_body(h_ref, b_ref, vn_ref, out_ref):
    ids = b_ref[0, 0, :]
    oh = (ids[:, None] == lax.broadcasted_iota(jnp.int32, (1, G), 1)
          ).astype(jnp.float32)
    out_ref[...] = h_ref[...] + lax.dot_general(
        oh, vn_ref[...], (((1,), (0,)), ((), ())),
        preferred_element_type=jnp.float32)


@jax.jit
def kernel(h, batch, vn_h, W1, bn_gamma, bn_beta, bn_mean, bn_var, W2):
    b3 = batch.reshape(GRID, 1, R)
    row2 = lambda v: v.reshape(1, DIM)

    full = pl.BlockSpec((G, DIM), lambda i: (0, 0))
    row = pl.BlockSpec((1, DIM), lambda i: (0, 0))
    hblk = pl.BlockSpec((R, DIM), lambda i: (i, 0))
    bblk = pl.BlockSpec((1, 1, R), lambda i: (i, 0, 0))

    vn_new = pl.pallas_call(
        _pool_mlp_body,
        grid=(GRID,),
        in_specs=[hblk, bblk, full, full, row, row, row, row, full],
        out_specs=full,
        out_shape=jax.ShapeDtypeStruct((G, DIM), jnp.float32),
        scratch_shapes=[pltpu.VMEM((G, DIM), jnp.float32)],
    )(h, b3, vn_h, W1, row2(bn_gamma), row2(bn_beta), row2(bn_mean),
      row2(bn_var), W2)

    h_new = pl.pallas_call(
        _broadcast_body,
        grid=(GRID,),
        in_specs=[hblk, bblk, full],
        out_specs=hblk,
        out_shape=jax.ShapeDtypeStruct((N, DIM), jnp.float32),
    )(h, b3, vn_new)
    return h_new
